# baseline (speedup 1.0000x reference)
import functools

import jax
import jax.numpy as jnp
from jax import lax
from jax.experimental import pallas as pl
from jax.experimental.pallas import tpu as pltpu
from jax.experimental.pallas import tpu_sc as plsc

N_ATOMS = 50000
N_EDGES = 1600000
N_FEATURES = 128
N_BASIS = 16
CUTOFF = 5.0

_DELTA = CUTOFF / (N_BASIS - 1)
_GAMMA = 1.0 / (2.0 * _DELTA * _DELTA)
_Q = N_EDGES // 128

_NC, _NS = 2, 16
_NW = _NC * _NS
_B_PER_W = 1568
_B_PAD = _NW * _B_PER_W
_N_CHUNK = 2
_CHUNK = _B_PER_W // _N_CHUNK


@functools.partial(
    pl.kernel,
    out_type=jax.ShapeDtypeStruct((_B_PAD, N_FEATURES), jnp.float32),
    mesh=plsc.VectorSubcoreMesh(core_axis_name="c", subcore_axis_name="s"),
    scratch_types=[
        pltpu.VMEM((_B_PER_W,), jnp.int32),
        pltpu.VMEM((_CHUNK, N_FEATURES), jnp.float32),
        pltpu.SemaphoreType.DMA,
    ],
)
def _gather_kernel(table_hbm, idx_hbm, out_hbm, idx_v, rows_v, sem):
    wid = lax.axis_index("s") * _NC + lax.axis_index("c")
    base = wid * _B_PER_W
    pltpu.sync_copy(idx_hbm.at[pl.ds(base, _B_PER_W)], idx_v)
    for c in range(_N_CHUNK):
        pltpu.async_copy(table_hbm.at[idx_v.at[pl.ds(c * _CHUNK, _CHUNK)]],
                         rows_v, sem).wait()
        pltpu.sync_copy(rows_v, out_hbm.at[pl.ds(base + c * _CHUNK, _CHUNK)])


_BA = 512


def _dir_body(disp_ref, dir_ref, dist_ref):
    d = disp_ref[...]
    l384 = lax.broadcasted_iota(jnp.int32, (384, 128), 0)
    e128 = lax.broadcasted_iota(jnp.int32, (384, 128), 1)
    m1 = (l384 // 3 == e128).astype(jnp.float32)
    n2 = jax.lax.dot(d * d, m1,
                     precision=jax.lax.Precision.HIGHEST) + 1e-12
    inv = lax.rsqrt(n2)
    dist_ref[...] = n2 * inv
    inv_exp = jax.lax.dot(inv, m1.T,
                          precision=jax.lax.Precision.HIGHEST)
    dir_ref[...] = d * inv_exp


_dir_call = pl.pallas_call(
    _dir_body,
    grid=(pl.cdiv(_Q, _BA),),
    in_specs=[pl.BlockSpec((_BA, 384), lambda i: (i, 0))],
    out_specs=[
        pl.BlockSpec((_BA, 384), lambda i: (i, 0)),
        pl.BlockSpec((_BA, 128), lambda i: (i, 0)),
    ],
    out_shape=[
        jax.ShapeDtypeStruct((_Q, 384), jnp.float32),
        jax.ShapeDtypeStruct((_Q, 128), jnp.float32),
    ],
)


_QB = 512


def _basis_body(dist_ref, out_ref):
    d = dist_ref[...]
    row = lax.broadcasted_iota(jnp.int32, (128, 128), 0)
    col = lax.broadcasted_iota(jnp.int32, (128, 128), 1)
    lane = lax.broadcasted_iota(jnp.int32, (1, 128), 1)
    centers = (lane % N_BASIS).astype(jnp.float32) * _DELTA
    for s in range(16):
        w = (row == 8 * s + col // 16).astype(jnp.float32)
        de = jax.lax.dot(d, w, precision=jax.lax.Precision.HIGHEST)
        cut = 0.5 * (jnp.cos((jnp.pi / CUTOFF) * de) + 1.0)
        cut = jnp.where(de < CUTOFF, cut, 0.0)
        diff = de - centers
        out_ref[:, s, :] = cut * jnp.exp(-_GAMMA * (diff * diff))


_basis_call = pl.pallas_call(
    _basis_body,
    grid=(pl.cdiv(_Q, _QB),),
    in_specs=[pl.BlockSpec((_QB, 128), lambda i: (i, 0))],
    out_specs=pl.BlockSpec((_QB, 16, 128), lambda i: (i, 0, 0)),
    out_shape=jax.ShapeDtypeStruct((_Q, 16, 128), jnp.float32),
)


def kernel(z, disp, node_table):
    zi = jnp.pad(z.astype(jnp.int32), (0, _B_PAD - N_ATOMS))
    atom_node = _gather_kernel(node_table, zi)[:N_ATOMS]
    disp_packed = disp.reshape(_Q, 384)
    dir_packed, dist_rows = _dir_call(disp_packed)
    dir_edge = dir_packed.reshape(N_EDGES, 3)
    dist_edge = _basis_call(dist_rows).reshape(N_EDGES, N_BASIS)
    force_node = jnp.zeros((N_ATOMS, 3, N_FEATURES), dtype=disp.dtype)
    disp_node = jnp.zeros((N_ATOMS, 3, N_FEATURES), dtype=disp.dtype)
    return (atom_node, force_node, disp_node, dir_edge, dist_edge)


# speedup vs baseline: 34.4732x; 34.4732x over previous
import functools

import jax
import jax.numpy as jnp
from jax import lax
from jax.experimental import pallas as pl
from jax.experimental.pallas import tpu as pltpu
from jax.experimental.pallas import tpu_sc as plsc

N_ATOMS = 50000
N_EDGES = 1600000
N_FEATURES = 128
N_BASIS = 16
CUTOFF = 5.0

_DELTA = CUTOFF / (N_BASIS - 1)
_GAMMA = 1.0 / (2.0 * _DELTA * _DELTA)

_NC, _NS = 2, 16
_NW = _NC * _NS
_B_PER_W = 1568
_B_PAD = _NW * _B_PER_W
_N_CHUNK = 2
_CHUNK = _B_PER_W // _N_CHUNK
_TAIL = N_ATOMS - (_NW - 1) * _B_PER_W - _CHUNK


@functools.partial(
    pl.kernel,
    out_type=jax.ShapeDtypeStruct((N_ATOMS, N_FEATURES), jnp.float32),
    mesh=plsc.VectorSubcoreMesh(core_axis_name="c", subcore_axis_name="s"),
    scratch_types=[
        pltpu.VMEM((_B_PER_W,), jnp.int32),
        pltpu.VMEM((_CHUNK, N_FEATURES), jnp.float32),
        pltpu.SemaphoreType.DMA,
    ],
)
def _gather_kernel(table_hbm, idx_hbm, out_hbm, idx_v, rows_v, sem):
    wid = lax.axis_index("s") * _NC + lax.axis_index("c")
    base = wid * _B_PER_W
    pltpu.sync_copy(idx_hbm.at[pl.ds(base, _B_PER_W)], idx_v)
    for c in range(_N_CHUNK):
        pltpu.async_copy(table_hbm.at[idx_v.at[pl.ds(c * _CHUNK, _CHUNK)]],
                         rows_v, sem).wait()
        if c < _N_CHUNK - 1:
            pltpu.sync_copy(rows_v, out_hbm.at[pl.ds(base + c * _CHUNK, _CHUNK)])
        else:
            @pl.when(wid < _NW - 1)
            def _full():
                pltpu.sync_copy(rows_v,
                                out_hbm.at[pl.ds(base + c * _CHUNK, _CHUNK)])

            @pl.when(wid == _NW - 1)
            def _tail():
                pltpu.sync_copy(rows_v.at[pl.ds(0, _TAIL)],
                                out_hbm.at[pl.ds(base + c * _CHUNK, _TAIL)])


_BT = 12800


def _edge_body(dispT_ref, dirT_ref, distT_ref):
    x = dispT_ref[0:1, :]
    y = dispT_ref[1:2, :]
    z = dispT_ref[2:3, :]
    n2 = x * x + y * y + z * z + 1e-12
    inv = lax.rsqrt(n2)
    dist = n2 * inv
    dirT_ref[0:1, :] = x * inv
    dirT_ref[1:2, :] = y * inv
    dirT_ref[2:3, :] = z * inv
    cut = 0.5 * (jnp.cos((jnp.pi / CUTOFF) * dist) + 1.0)
    cut = jnp.where(dist < CUTOFF, cut, 0.0)
    centers = lax.broadcasted_iota(jnp.int32, (N_BASIS, _BT), 0).astype(
        jnp.float32) * _DELTA
    db = jnp.broadcast_to(dist, (N_BASIS, _BT))
    cb = jnp.broadcast_to(cut, (N_BASIS, _BT))
    diff = db - centers
    distT_ref[...] = cb * jnp.exp(-_GAMMA * (diff * diff))


_edge_call = pl.pallas_call(
    _edge_body,
    grid=(N_EDGES // _BT,),
    in_specs=[pl.BlockSpec((3, _BT), lambda i: (0, i))],
    out_specs=[
        pl.BlockSpec((3, _BT), lambda i: (0, i)),
        pl.BlockSpec((N_BASIS, _BT), lambda i: (0, i)),
    ],
    out_shape=[
        jax.ShapeDtypeStruct((3, N_EDGES), jnp.float32),
        jax.ShapeDtypeStruct((N_BASIS, N_EDGES), jnp.float32),
    ],
)


def kernel(z, disp, node_table):
    zi = jnp.pad(z.astype(jnp.int32), (0, _B_PAD - N_ATOMS))
    atom_node = _gather_kernel(node_table, zi)
    dirT, distT = _edge_call(disp.T)
    dir_edge = dirT.T
    dist_edge = distT.T
    force_node = jnp.zeros((N_ATOMS, 3, N_FEATURES), dtype=disp.dtype)
    disp_node = jnp.zeros((N_ATOMS, 3, N_FEATURES), dtype=disp.dtype)
    return (atom_node, force_node, disp_node, dir_edge, dist_edge)


# speedup vs baseline: 40.2783x; 1.1684x over previous
import functools

import jax
import jax.numpy as jnp
from jax import lax
from jax.experimental import pallas as pl
from jax.experimental.pallas import tpu as pltpu
from jax.experimental.pallas import tpu_sc as plsc

N_ATOMS = 50000
N_EDGES = 1600000
N_FEATURES = 128
N_BASIS = 16
CUTOFF = 5.0

_DELTA = CUTOFF / (N_BASIS - 1)
_GAMMA = 1.0 / (2.0 * _DELTA * _DELTA)

_NC, _NS = 2, 16
_NW = _NC * _NS
_B_PER_W = 1568
_B_PAD = _NW * _B_PER_W
_N_CHUNK = 2
_CHUNK = _B_PER_W // _N_CHUNK
_TAIL = N_ATOMS - (_NW - 1) * _B_PER_W - _CHUNK


@functools.partial(
    pl.kernel,
    out_type=jax.ShapeDtypeStruct((N_ATOMS, N_FEATURES), jnp.float32),
    mesh=plsc.VectorSubcoreMesh(core_axis_name="c", subcore_axis_name="s"),
    scratch_types=[
        pltpu.VMEM((_B_PER_W,), jnp.int32),
        pltpu.VMEM((_CHUNK, N_FEATURES), jnp.float32),
        pltpu.SemaphoreType.DMA,
    ],
)
def _gather_kernel(table_hbm, idx_hbm, out_hbm, idx_v, rows_v, sem):
    wid = lax.axis_index("s") * _NC + lax.axis_index("c")
    base = wid * _B_PER_W
    pltpu.sync_copy(idx_hbm.at[pl.ds(base, _B_PER_W)], idx_v)
    for c in range(_N_CHUNK):
        pltpu.async_copy(table_hbm.at[idx_v.at[pl.ds(c * _CHUNK, _CHUNK)]],
                         rows_v, sem).wait()
        if c < _N_CHUNK - 1:
            pltpu.sync_copy(rows_v, out_hbm.at[pl.ds(base + c * _CHUNK, _CHUNK)])
        else:
            @pl.when(wid < _NW - 1)
            def _full():
                pltpu.sync_copy(rows_v,
                                out_hbm.at[pl.ds(base + c * _CHUNK, _CHUNK)])

            @pl.when(wid == _NW - 1)
            def _tail():
                pltpu.sync_copy(rows_v.at[pl.ds(0, _TAIL)],
                                out_hbm.at[pl.ds(base + c * _CHUNK, _TAIL)])


_BT = 32000
_CENTERS = tuple(float(k) * _DELTA for k in range(N_BASIS))


def _edge_body(dispT_ref, dirT_ref, distT_ref):
    x = dispT_ref[0:1, :]
    y = dispT_ref[1:2, :]
    z = dispT_ref[2:3, :]
    n2 = x * x + y * y + z * z + 1e-12
    inv = lax.rsqrt(n2)
    dist = n2 * inv
    dirT_ref[0:1, :] = x * inv
    dirT_ref[1:2, :] = y * inv
    dirT_ref[2:3, :] = z * inv
    cut = 0.5 * (jnp.cos((jnp.pi / CUTOFF) * dist) + 1.0)
    cut = jnp.where(dist < CUTOFF, cut, 0.0)
    centers = lax.broadcasted_iota(jnp.int32, (N_BASIS, 1), 0).astype(
        jnp.float32) * _DELTA
    diff = dist - centers
    distT_ref[...] = cut * jnp.exp(-_GAMMA * (diff * diff))


_edge_call = pl.pallas_call(
    _edge_body,
    grid=(N_EDGES // _BT,),
    in_specs=[pl.BlockSpec((3, _BT), lambda i: (0, i))],
    out_specs=[
        pl.BlockSpec((3, _BT), lambda i: (0, i)),
        pl.BlockSpec((N_BASIS, _BT), lambda i: (0, i)),
    ],
    out_shape=[
        jax.ShapeDtypeStruct((3, N_EDGES), jnp.float32),
        jax.ShapeDtypeStruct((N_BASIS, N_EDGES), jnp.float32),
    ],
)


def kernel(z, disp, node_table):
    zi = jnp.pad(z.astype(jnp.int32), (0, _B_PAD - N_ATOMS))
    atom_node = _gather_kernel(node_table, zi)
    dirT, distT = _edge_call(disp.T)
    dir_edge = dirT.T
    dist_edge = distT.T
    force_node = jnp.zeros((N_ATOMS, 3, N_FEATURES), dtype=disp.dtype)
    disp_node = jnp.zeros((N_ATOMS, 3, N_FEATURES), dtype=disp.dtype)
    return (atom_node, force_node, disp_node, dir_edge, dist_edge)
